# baseline (speedup 1.0000x reference)
import functools

import jax
import jax.numpy as jnp
from jax import lax
from jax.experimental import pallas as pl
from jax.experimental.pallas import tpu as pltpu
from jax.experimental.pallas import tpu_sc as plsc

_MAX_SLIP = 0.4
_MAX_GUESS = 0.4
_LN2 = 0.6931471805599453

_B = 16384
_H = 128
_NC = 2
_NS = 16
_NW = _NC * _NS
_BPW = _B // _NW
_CH = 64
_NCHUNK = _BPW // _CH
_KB = 4
_L = 16


def _vlog(x):
    bits = plsc.bitcast(x, jnp.int32)
    e = ((bits >> 23) & 0xFF) - 127
    m = plsc.bitcast((bits & 0x7FFFFF) | (127 << 23), jnp.float32)
    big = m > 1.4142135
    m = jnp.where(big, m * 0.5, m)
    ef = jnp.where(big, e + 1, e).astype(jnp.float32)
    z = (m - 1.0) / (m + 1.0)
    z2 = z * z
    p = z * (2.0 + z2 * (0.6666667 + z2 * (0.4 + z2 * (0.2857143 + z2 * 0.2222222))))
    return ef * _LN2 + p


def _body(user_ref, item_ref, knowledge_ref, theta_ref, slip_ref, guess_ref,
          out_ref, uidx, iidx, tbufs, kbufs, sraw, graw, cvals, obuf,
          semt, semk, semsg):
    wid = lax.axis_index("s") * _NC + lax.axis_index("c")
    rowbase = wid * _NCHUNK
    ebase = wid * _BPW

    pltpu.sync_copy(user_ref.at[pl.ds(rowbase, _NCHUNK)], uidx)

    def kn_copy(k):
        return pltpu.make_async_copy(
            knowledge_ref.at[pl.ds(ebase + k * _CH, _CH)], kbufs.at[k % _KB],
            semk.at[k % _KB])

    tcop = [pltpu.make_async_copy(theta_ref.at[uidx.at[k]], tbufs.at[k],
                                  semt.at[k])
            for k in range(_NCHUNK)]
    kcop = [kn_copy(k) for k in range(_NCHUNK)]
    for k in range(_NCHUNK):
        tcop[k].start()
        if k < _KB:
            kcop[k].start()

    pltpu.sync_copy(item_ref.at[pl.ds(rowbase, _NCHUNK)], iidx)
    sg_copies = []
    for k in range(_NCHUNK):
        c1 = pltpu.make_async_copy(slip_ref.at[iidx.at[k]], sraw.at[k], semsg)
        c1.start()
        c2 = pltpu.make_async_copy(guess_ref.at[iidx.at[k]], graw.at[k], semsg)
        c2.start()
        sg_copies += [c1, c2]

    lane = lax.iota(jnp.int32, _L)

    for k in range(_NCHUNK):
        with jax.named_scope(f"wait{k}"):
            tcop[k].wait()
            kcop[k].wait()
        tb, kb = tbufs.at[k], kbufs.at[k % _KB]

        def group(g, _, tb=tb, kb=kb, k=k):
            rows = g * _L + lane

            def hblock(hb, carry):
                cols = carry[0]
                accs = list(carry[1:])
                for h in range(32):
                    t = plsc.load_gather(tb, [rows, cols])
                    kn = plsc.load_gather(kb, [rows, cols])
                    j = h % 4
                    accs[j] = jnp.where(t <= 0.0, accs[j] + kn, accs[j])
                    cols = (cols + 1) & (_H - 1)
                return (cols, *accs)

            z = jnp.zeros((_L,), jnp.float32)
            carry = lax.fori_loop(0, _H // 32, hblock, (lane, z, z, z, z))
            cvals[k, pl.ds(g * _L, _L)] = (carry[1] + carry[2]) + (carry[3] + carry[4])
            return 0

        with jax.named_scope(f"comp{k}"):
            lax.fori_loop(0, _CH // _L, group, 0)

        if k + _KB < _NCHUNK:
            kcop[k + _KB].start()

    for c in sg_copies:
        c.wait()

    for k in range(_NCHUNK):
        def pgroup(j, _, k=k):
            c = cvals[k, pl.ds(j * _L, _L)].astype(jnp.int32)
            sr = sraw[k, pl.ds(j * _L, _L)]
            gr = graw[k, pl.ds(j * _L, _L)]
            bs = 1.0 + jnp.exp(-sr)
            bg = 1.0 + jnp.exp(-gr)
            g = _MAX_GUESS / bg
            one_m_s = 1.0 - _MAX_SLIP / bs
            q = one_m_s * (2.5 * bg)
            nbits = (127 - c) << 23
            n = jnp.where(c < 127, plsc.bitcast(nbits, jnp.float32), 0.0)
            r = g * jnp.exp(n * _vlog(q))
            obuf[k, pl.ds(j * _L, _L)] = r
            return 0

        with jax.named_scope(f"phase2_{k}"):
            lax.fori_loop(0, _CH // _L, pgroup, 0)

    pltpu.sync_copy(obuf, out_ref.at[pl.ds(rowbase, _NCHUNK)])


_sc_kernel = functools.partial(
    pl.kernel,
    mesh=plsc.VectorSubcoreMesh(core_axis_name="c", subcore_axis_name="s"),
    out_type=jax.ShapeDtypeStruct((_B // _CH, _CH), jnp.float32),
    compiler_params=pltpu.CompilerParams(needs_layout_passes=False),
    scratch_types=[
        pltpu.VMEM((_NCHUNK, _CH), jnp.int32),
        pltpu.VMEM((_NCHUNK, _CH), jnp.int32),
        pltpu.VMEM((_NCHUNK, _CH, _H), jnp.float32),
        pltpu.VMEM((_KB, _CH, _H), jnp.float32),
        pltpu.VMEM((_NCHUNK, _CH), jnp.float32),
        pltpu.VMEM((_NCHUNK, _CH), jnp.float32),
        pltpu.VMEM((_NCHUNK, _CH), jnp.float32),
        pltpu.VMEM((_NCHUNK, _CH), jnp.float32),
        pltpu.SemaphoreType.DMA((_NCHUNK,)),
        pltpu.SemaphoreType.DMA((_KB,)),
        pltpu.SemaphoreType.DMA,
    ],
)(_body)


def kernel(user, item, knowledge, theta_table, slip_table, guess_table):
    user2d = user.astype(jnp.int32).reshape(_B // _CH, _CH)
    item2d = item.astype(jnp.int32).reshape(_B // _CH, _CH)
    out2d = _sc_kernel(user2d, item2d, knowledge, theta_table,
                       slip_table.reshape(-1), guess_table.reshape(-1))
    return out2d.reshape(-1)


# speedup vs baseline: 1.2326x; 1.2326x over previous
import functools

import jax
import jax.numpy as jnp
from jax import lax
from jax.experimental import pallas as pl
from jax.experimental.pallas import tpu as pltpu
from jax.experimental.pallas import tpu_sc as plsc

_MAX_SLIP = 0.4
_MAX_GUESS = 0.4
_LN2 = 0.6931471805599453

_B = 16384
_H = 128
_NC = 2
_NS = 16
_NW = _NC * _NS
_BPW = _B // _NW
_CH = 128
_NCHUNK = _BPW // _CH
_KB = 3
_L = 16


def _vlog(x):
    bits = plsc.bitcast(x, jnp.int32)
    e = ((bits >> 23) & 0xFF) - 127
    m = plsc.bitcast((bits & 0x7FFFFF) | (127 << 23), jnp.float32)
    big = m > 1.4142135
    m = jnp.where(big, m * 0.5, m)
    ef = jnp.where(big, e + 1, e).astype(jnp.float32)
    z = (m - 1.0) / (m + 1.0)
    z2 = z * z
    p = z * (2.0 + z2 * (0.6666667 + z2 * (0.4 + z2 * (0.2857143 + z2 * 0.2222222))))
    return ef * _LN2 + p


def _body(user_ref, item_ref, knowledge_ref, theta_ref, slip_ref, guess_ref,
          out_ref, uidx, iidx, tbufs, kbufs, sraw, graw, cvals, obuf,
          semt, semk, semsg):
    wid = lax.axis_index("s") * _NC + lax.axis_index("c")
    rowbase = wid * _NCHUNK
    ebase = wid * _BPW

    pltpu.sync_copy(user_ref.at[pl.ds(rowbase, _NCHUNK)], uidx)

    def kn_copy(k):
        return pltpu.make_async_copy(
            knowledge_ref.at[pl.ds(ebase + k * _CH, _CH)], kbufs.at[k % _KB],
            semk.at[k % _KB])

    tcop = [pltpu.make_async_copy(theta_ref.at[uidx.at[k]], tbufs.at[k],
                                  semt.at[k])
            for k in range(_NCHUNK)]
    kcop = [kn_copy(k) for k in range(_NCHUNK)]
    for k in range(_NCHUNK):
        tcop[k].start()
        if k < _KB:
            kcop[k].start()

    pltpu.sync_copy(item_ref.at[pl.ds(rowbase, _NCHUNK)], iidx)
    sg_copies = []
    for k in range(_NCHUNK):
        c1 = pltpu.make_async_copy(slip_ref.at[iidx.at[k]], sraw.at[k], semsg)
        c1.start()
        c2 = pltpu.make_async_copy(guess_ref.at[iidx.at[k]], graw.at[k], semsg)
        c2.start()
        sg_copies += [c1, c2]

    lane = lax.iota(jnp.int32, _L)

    for k in range(_NCHUNK):
        with jax.named_scope(f"wait{k}"):
            tcop[k].wait()
            kcop[k].wait()
        tb, kb = tbufs.at[k], kbufs.at[k % _KB]

        def group(g, _, tb=tb, kb=kb, k=k):
            rows = g * _L + lane

            def hblock(hb, carry):
                cols = carry[0]
                accs = list(carry[1:])
                for h in range(32):
                    t = plsc.load_gather(tb, [rows, cols])
                    kn = plsc.load_gather(kb, [rows, cols])
                    j = h % 4
                    accs[j] = jnp.where(t <= 0.0, accs[j] + kn, accs[j])
                    cols = (cols + 1) & (_H - 1)
                return (cols, *accs)

            z = jnp.zeros((_L,), jnp.float32)
            carry = lax.fori_loop(0, _H // 32, hblock, (lane, z, z, z, z))
            cvals[k, pl.ds(g * _L, _L)] = (carry[1] + carry[2]) + (carry[3] + carry[4])
            return 0

        with jax.named_scope(f"comp{k}"):
            lax.fori_loop(0, _CH // _L, group, 0)

        if k + _KB < _NCHUNK:
            kcop[k + _KB].start()

    for c in sg_copies:
        c.wait()

    for k in range(_NCHUNK):
        def pgroup(j, _, k=k):
            c = cvals[k, pl.ds(j * _L, _L)].astype(jnp.int32)
            sr = sraw[k, pl.ds(j * _L, _L)]
            gr = graw[k, pl.ds(j * _L, _L)]
            bs = 1.0 + jnp.exp(-sr)
            bg = 1.0 + jnp.exp(-gr)
            g = _MAX_GUESS / bg
            one_m_s = 1.0 - _MAX_SLIP / bs
            q = one_m_s * (2.5 * bg)
            nbits = (127 - c) << 23
            n = jnp.where(c < 127, plsc.bitcast(nbits, jnp.float32), 0.0)
            r = g * jnp.exp(n * _vlog(q))
            obuf[k, pl.ds(j * _L, _L)] = r
            return 0

        with jax.named_scope(f"phase2_{k}"):
            lax.fori_loop(0, _CH // _L, pgroup, 0)

    pltpu.sync_copy(obuf, out_ref.at[pl.ds(rowbase, _NCHUNK)])


_sc_kernel = functools.partial(
    pl.kernel,
    mesh=plsc.VectorSubcoreMesh(core_axis_name="c", subcore_axis_name="s"),
    out_type=jax.ShapeDtypeStruct((_B // _CH, _CH), jnp.float32),
    compiler_params=pltpu.CompilerParams(needs_layout_passes=False),
    scratch_types=[
        pltpu.VMEM((_NCHUNK, _CH), jnp.int32),
        pltpu.VMEM((_NCHUNK, _CH), jnp.int32),
        pltpu.VMEM((_NCHUNK, _CH, _H), jnp.float32),
        pltpu.VMEM((_KB, _CH, _H), jnp.float32),
        pltpu.VMEM((_NCHUNK, _CH), jnp.float32),
        pltpu.VMEM((_NCHUNK, _CH), jnp.float32),
        pltpu.VMEM((_NCHUNK, _CH), jnp.float32),
        pltpu.VMEM((_NCHUNK, _CH), jnp.float32),
        pltpu.SemaphoreType.DMA((_NCHUNK,)),
        pltpu.SemaphoreType.DMA((_KB,)),
        pltpu.SemaphoreType.DMA,
    ],
)(_body)


def kernel(user, item, knowledge, theta_table, slip_table, guess_table):
    user2d = user.astype(jnp.int32).reshape(_B // _CH, _CH)
    item2d = item.astype(jnp.int32).reshape(_B // _CH, _CH)
    out2d = _sc_kernel(user2d, item2d, knowledge, theta_table,
                       slip_table.reshape(-1), guess_table.reshape(-1))
    return out2d.reshape(-1)
